# baseline (speedup 1.0000x reference)
import functools

import jax
import jax.numpy as jnp
from jax import lax
from jax.experimental import pallas as pl
from jax.experimental.pallas import tpu as pltpu
from jax.experimental.pallas import tpu_sc as plsc

_B = 16384
_D = 64
_NB = 1000000
_NC = 2
_NS = 16
_NW = _NC * _NS
_BPW = _B // _NW
_NG = _BPW // 16
_R = 6
_BM = 2048


def _sc_gather(sr_all, pos_all, tab_T):
    mesh = plsc.VectorSubcoreMesh(core_axis_name="c", subcore_axis_name="s")

    @functools.partial(
        pl.kernel,
        out_type=jax.ShapeDtypeStruct((_B, 128), jnp.float32),
        mesh=mesh,
        scratch_types=[
            pltpu.VMEM((_BPW,), jnp.int32),
            pltpu.VMEM((4, 128), jnp.int32),
            pltpu.SMEM((_BPW,), jnp.int32),
            pltpu.VMEM((_R, 64, 128), jnp.float32),
            pltpu.VMEM((_BPW, 128), jnp.float32),
            pltpu.SemaphoreType.DMA,
            pltpu.SemaphoreType.DMA((_R,)),
            pltpu.SemaphoreType.DMA,
        ],
        compiler_params=pltpu.CompilerParams(needs_layout_passes=False),
    )
    def gather(sr_hbm, pos_hbm, tab, out,
               sr_v, pos_v, ucols_s, colbuf, rows, sem_i, sem_c, sem_o):
        wid = lax.axis_index("s") * _NC + lax.axis_index("c")
        if True:
            pltpu.sync_copy(sr_hbm.at[wid], sr_v)
            pltpu.sync_copy(pos_hbm.at[wid], pos_v)

            def grp_a(g, carry):
                n_u, prev = carry
                cvec = lax.shift_right_logical(sr_v[pl.ds(g * 16, 16)], 7)
                for q in range(16):
                    c = cvec[q]
                    if q == 0:
                        chg = jnp.logical_or(g == 0, c != prev)
                    else:
                        chg = c != prev
                    @pl.when(chg)
                    def _():
                        ucols_s[n_u] = c
                    n_u = n_u + chg.astype(jnp.int32)
                    prev = c
                return n_u, prev

            n_u, _ = pl.loop(0, _NG, init_carry=(jnp.int32(0), jnp.int32(0)))(
                grp_a)

            for s in range(_R):
                @pl.when(s < n_u)
                def _():
                    c = ucols_s[s]
                    pltpu.async_copy(
                        tab.at[:, pl.ds(pl.multiple_of(c * 128, 128), 128)],
                        colbuf.at[s], sem_c.at[s])

            def grp_b(g, carry):
                u, prev = carry
                vec = sr_v[pl.ds(g * 16, 16)]
                cvec = lax.shift_right_logical(vec, 7)
                lvec = lax.bitwise_and(vec, 127)
                for q in range(16):
                    c = cvec[q]
                    l = lvec[q]
                    if q == 0:
                        chg = jnp.logical_or(
                            jnp.logical_and(g == 0, u < 0), c != prev)
                    else:
                        chg = c != prev
                    u = u + chg.astype(jnp.int32)

                    @pl.when(chg)
                    def _():
                        @pl.when(jnp.logical_and(u > 0, u - 1 + _R < n_u))
                        def _():
                            c2 = ucols_s[u - 1 + _R]
                            pltpu.async_copy(
                                tab.at[:, pl.ds(
                                    pl.multiple_of(c2 * 128, 128), 128)],
                                colbuf.at[lax.rem(u - 1, _R)],
                                sem_c.at[lax.rem(u - 1, _R)])
                        slot_w = lax.rem(u, _R)
                        pltpu.make_async_copy(
                            tab.at[:, pl.ds(0, 128)], colbuf.at[slot_w],
                            sem_c.at[slot_w]).wait()

                    slot = lax.rem(u, _R)
                    m = g * 16 + q
                    lsplat = jnp.full((16,), l, jnp.int32)
                    for j in range(4):
                        d_idx = lax.iota(jnp.int32, 16) + (16 * j)
                        vals = plsc.load_gather(
                            colbuf.at[slot], [d_idx, lsplat])
                        rows[m, pl.ds(16 * j, 16)] = vals
                    prev = c
                return u, prev

            pl.loop(0, _NG, init_carry=(jnp.int32(-1), jnp.int32(0)))(grp_b)

            cps = []
            for j in range(4):
                cps.append(pltpu.async_copy(
                    rows.at[pl.ds(j * 128, 128)], out.at[pos_v.at[j]], sem_o))
            for cp in cps:
                cp.wait()

    return gather(sr_all, pos_all, tab_T)


def _tc_dense(u, i, c, lwu, lwi, lwc, cku, cki, ckc,
              w1u, w1i, w1c, b1, w2, b2, w3, b3, w4, b4, w5, bias):
    f32 = jnp.float32

    def body(u_ref, i_ref, c_ref, lwu_ref, lwi_ref, lwc_ref,
             cku_ref, cki_ref, ckc_ref, w1u_ref, w1i_ref, w1c_ref, b1_ref,
             w2_ref, b2_ref, w3_ref, b3_ref, w4_ref, b4_ref, w5_ref,
             bias_ref, out_ref):
        uu = u_ref[:, :64]
        ii = i_ref[:, :64]
        cc = c_ref[:, :64]
        lin = jnp.sum(uu * lwu_ref[...] + ii * lwi_ref[...]
                      + cc * lwc_ref[...], axis=1, keepdims=True)
        xk = (jnp.dot(uu, cku_ref[...], preferred_element_type=f32)
              + jnp.dot(ii, cki_ref[...], preferred_element_type=f32)
              + jnp.dot(cc, ckc_ref[...], preferred_element_type=f32))
        a_sum = jnp.sum(xk * xk, axis=1, keepdims=True)
        bt = (jnp.dot(uu * uu, jnp.square(cku_ref[...]),
                      preferred_element_type=f32)
              + jnp.dot(ii * ii, jnp.square(cki_ref[...]),
                        preferred_element_type=f32)
              + jnp.dot(cc * cc, jnp.square(ckc_ref[...]),
                        preferred_element_type=f32))
        b_sum = jnp.sum(bt, axis=1, keepdims=True)
        fm = lin + (0.5 / 128.0) * (a_sum - b_sum)
        h = jax.nn.relu(jnp.dot(uu, w1u_ref[...], preferred_element_type=f32)
                        + jnp.dot(ii, w1i_ref[...], preferred_element_type=f32)
                        + jnp.dot(cc, w1c_ref[...], preferred_element_type=f32)
                        + b1_ref[...])
        h = jax.nn.relu(jnp.dot(h, w2_ref[...], preferred_element_type=f32)
                        + b2_ref[...])
        h = jax.nn.relu(jnp.dot(h, w3_ref[...], preferred_element_type=f32)
                        + b3_ref[...])
        h = jax.nn.relu(jnp.dot(h, w4_ref[...], preferred_element_type=f32)
                        + b4_ref[...])
        dnn = jnp.sum(h * w5_ref[...], axis=1, keepdims=True)
        out_ref[...] = jax.nn.sigmoid(fm + dnn + bias_ref[...])

    grid = (_B // _BM,)
    row_spec = pl.BlockSpec((_BM, 128), lambda m: (m, 0))

    def full(x):
        return pl.BlockSpec(x.shape, lambda m: (0,) * x.ndim)

    weights = (lwu, lwi, lwc, cku, cki, ckc, w1u, w1i, w1c, b1,
               w2, b2, w3, b3, w4, b4, w5, bias)
    return pl.pallas_call(
        body,
        grid=grid,
        in_specs=[row_spec, row_spec, row_spec] + [full(w) for w in weights],
        out_specs=pl.BlockSpec((_BM, 1), lambda m: (m, 0)),
        out_shape=jax.ShapeDtypeStruct((_B, 1), jnp.float32),
        compiler_params=pltpu.CompilerParams(
            dimension_semantics=("arbitrary",)),
    )(u, i, c, *weights)


def kernel(user_id, item_id, item_catalog, user_table, item_table,
           catalog_table, lin_W, lin_b, cross_K, W1, b1, W2, b2, W3, b3,
           W4, b4, W5, b5):
    ids = [user_id.astype(jnp.int32) % _NB, item_id.astype(jnp.int32) % _NB,
           item_catalog.astype(jnp.int32) % _NB]
    outs = []
    for r, tab in zip(ids, (user_table, item_table, catalog_table)):
        perm = jnp.argsort(r).astype(jnp.int32)
        sr = jnp.take(r, perm).reshape(_NW, _BPW)
        pos = perm.reshape(_NW, 4, 128)
        outs.append(_sc_gather(sr, pos, tab.T))
    u_rows, i_rows, c_rows = outs
    lwu = lin_W[0:64].reshape(1, _D)
    lwi = lin_W[64:128].reshape(1, _D)
    lwc = lin_W[128:192].reshape(1, _D)
    cku, cki, ckc = cross_K[0:64], cross_K[64:128], cross_K[128:192]
    w1u, w1i, w1c = W1[0:64], W1[64:128], W1[128:192]
    bias = (lin_b + b5).reshape(1, 1)
    return _tc_dense(u_rows, i_rows, c_rows, lwu, lwi, lwc, cku, cki, ckc,
                     w1u, w1i, w1c, b1.reshape(1, 256), W2, b2.reshape(1, 128),
                     W3, b3.reshape(1, 64), W4, b4.reshape(1, 32),
                     W5.reshape(1, 32), bias)


# speedup vs baseline: 1.0658x; 1.0658x over previous
import functools

import jax
import jax.numpy as jnp
from jax import lax
from jax.experimental import pallas as pl
from jax.experimental.pallas import tpu as pltpu
from jax.experimental.pallas import tpu_sc as plsc

_B = 16384
_D = 64
_NB = 1000000
_NC = 2
_NS = 16
_NW = _NC * _NS
_BPW = _B // _NW
_NG = _BPW // 16
_R = 8
_BM = 2048


def _sc_gather(sr_all, pos_all, ut_T, it_T, ct_T):
    mesh = plsc.VectorSubcoreMesh(core_axis_name="c", subcore_axis_name="s")

    @functools.partial(
        pl.kernel,
        out_type=(jax.ShapeDtypeStruct((_B, 128), jnp.float32),) * 3,
        mesh=mesh,
        scratch_types=[
            pltpu.VMEM((_BPW,), jnp.int32),
            pltpu.VMEM((4, 128), jnp.int32),
            pltpu.SMEM((_BPW,), jnp.int32),
            pltpu.VMEM((_R, 64, 128), jnp.float32),
            pltpu.VMEM((_BPW // 2, 128), jnp.float32),
            pltpu.SemaphoreType.DMA,
            pltpu.SemaphoreType.DMA((_R,)),
            pltpu.SemaphoreType.DMA,
        ],
        compiler_params=pltpu.CompilerParams(needs_layout_passes=False),
    )
    def gather(sr_hbm, pos_hbm, ut_hbm, it_hbm, ct_hbm, u_out, i_out, c_out,
               sr_v, pos_v, ucols_s, colbuf, rows, sem_i, sem_c, sem_o):
        wid = lax.axis_index("s") * _NC + lax.axis_index("c")
        for t, (tab, out) in enumerate(
            ((ut_hbm, u_out), (it_hbm, i_out), (ct_hbm, c_out))):
            pltpu.sync_copy(sr_hbm.at[t * _NW + wid], sr_v)
            pltpu.sync_copy(pos_hbm.at[t * _NW + wid], pos_v)

            def grp_a(g, carry):
                n_u, prev = carry
                cvec = lax.shift_right_logical(sr_v[pl.ds(g * 16, 16)], 7)
                for q in range(16):
                    c = cvec[q]
                    if q == 0:
                        chg = jnp.logical_or(g == 0, c != prev)
                    else:
                        chg = c != prev
                    @pl.when(chg)
                    def _():
                        ucols_s[n_u] = c
                    n_u = n_u + chg.astype(jnp.int32)
                    prev = c
                return n_u, prev

            n_u, _ = pl.loop(0, _NG, init_carry=(jnp.int32(0), jnp.int32(0)))(
                grp_a)

            for s in range(_R):
                @pl.when(s < n_u)
                def _():
                    c = ucols_s[s]
                    pltpu.async_copy(
                        tab.at[:, pl.ds(pl.multiple_of(c * 128, 128), 128)],
                        colbuf.at[s], sem_c.at[s])

            def mk_grp_b(mbase):
                def grp_b(g, carry):
                    u, prev = carry
                    vec = sr_v[pl.ds(g * 16, 16)]
                    cvec = lax.shift_right_logical(vec, 7)
                    lvec = lax.bitwise_and(vec, 127)
                    for q in range(16):
                        c = cvec[q]
                        l = lvec[q]
                        if q == 0:
                            chg = jnp.logical_or(
                                jnp.logical_and(g == 0, u < 0), c != prev)
                        else:
                            chg = c != prev
                        u = u + chg.astype(jnp.int32)

                        @pl.when(chg)
                        def _():
                            @pl.when(jnp.logical_and(u > 0, u - 1 + _R < n_u))
                            def _():
                                c2 = ucols_s[u - 1 + _R]
                                pltpu.async_copy(
                                    tab.at[:, pl.ds(
                                        pl.multiple_of(c2 * 128, 128), 128)],
                                    colbuf.at[lax.rem(u - 1, _R)],
                                    sem_c.at[lax.rem(u - 1, _R)])
                            slot_w = lax.rem(u, _R)
                            pltpu.make_async_copy(
                                tab.at[:, pl.ds(0, 128)], colbuf.at[slot_w],
                                sem_c.at[slot_w]).wait()

                        slot = lax.rem(u, _R)
                        m = g * 16 + q - mbase
                        lsplat = jnp.full((16,), l, jnp.int32)
                        for j in range(4):
                            d_idx = lax.iota(jnp.int32, 16) + (16 * j)
                            vals = plsc.load_gather(
                                colbuf.at[slot], [d_idx, lsplat])
                            rows[m, pl.ds(16 * j, 16)] = vals
                        prev = c
                    return u, prev
                return grp_b

            carry = (jnp.int32(-1), jnp.int32(0))
            for half in range(2):
                carry = pl.loop(half * (_NG // 2), (half + 1) * (_NG // 2),
                                init_carry=carry)(
                    mk_grp_b(half * (_BPW // 2)))
                cps = []
                for j in range(2):
                    cps.append(pltpu.async_copy(
                        rows.at[pl.ds(j * 128, 128)],
                        out.at[pos_v.at[half * 2 + j]], sem_o))
                for cp in cps:
                    cp.wait()

    return gather(sr_all, pos_all, ut_T, it_T, ct_T)


def _tc_dense(u, i, c, lwu, lwi, lwc, cku, cki, ckc,
              w1u, w1i, w1c, b1, w2, b2, w3, b3, w4, b4, w5, bias):
    f32 = jnp.float32

    def body(u_ref, i_ref, c_ref, lwu_ref, lwi_ref, lwc_ref,
             cku_ref, cki_ref, ckc_ref, w1u_ref, w1i_ref, w1c_ref, b1_ref,
             w2_ref, b2_ref, w3_ref, b3_ref, w4_ref, b4_ref, w5_ref,
             bias_ref, out_ref):
        uu = u_ref[:, :64]
        ii = i_ref[:, :64]
        cc = c_ref[:, :64]
        lin = jnp.sum(uu * lwu_ref[...] + ii * lwi_ref[...]
                      + cc * lwc_ref[...], axis=1, keepdims=True)
        xk = (jnp.dot(uu, cku_ref[...], preferred_element_type=f32)
              + jnp.dot(ii, cki_ref[...], preferred_element_type=f32)
              + jnp.dot(cc, ckc_ref[...], preferred_element_type=f32))
        a_sum = jnp.sum(xk * xk, axis=1, keepdims=True)
        bt = (jnp.dot(uu * uu, jnp.square(cku_ref[...]),
                      preferred_element_type=f32)
              + jnp.dot(ii * ii, jnp.square(cki_ref[...]),
                        preferred_element_type=f32)
              + jnp.dot(cc * cc, jnp.square(ckc_ref[...]),
                        preferred_element_type=f32))
        b_sum = jnp.sum(bt, axis=1, keepdims=True)
        fm = lin + (0.5 / 128.0) * (a_sum - b_sum)
        h = jax.nn.relu(jnp.dot(uu, w1u_ref[...], preferred_element_type=f32)
                        + jnp.dot(ii, w1i_ref[...], preferred_element_type=f32)
                        + jnp.dot(cc, w1c_ref[...], preferred_element_type=f32)
                        + b1_ref[...])
        h = jax.nn.relu(jnp.dot(h, w2_ref[...], preferred_element_type=f32)
                        + b2_ref[...])
        h = jax.nn.relu(jnp.dot(h, w3_ref[...], preferred_element_type=f32)
                        + b3_ref[...])
        h = jax.nn.relu(jnp.dot(h, w4_ref[...], preferred_element_type=f32)
                        + b4_ref[...])
        dnn = jnp.sum(h * w5_ref[...], axis=1, keepdims=True)
        out_ref[...] = jax.nn.sigmoid(fm + dnn + bias_ref[...])

    grid = (_B // _BM,)
    row_spec = pl.BlockSpec((_BM, 128), lambda m: (m, 0))

    def full(x):
        return pl.BlockSpec(x.shape, lambda m: (0,) * x.ndim)

    weights = (lwu, lwi, lwc, cku, cki, ckc, w1u, w1i, w1c, b1,
               w2, b2, w3, b3, w4, b4, w5, bias)
    return pl.pallas_call(
        body,
        grid=grid,
        in_specs=[row_spec, row_spec, row_spec] + [full(w) for w in weights],
        out_specs=pl.BlockSpec((_BM, 1), lambda m: (m, 0)),
        out_shape=jax.ShapeDtypeStruct((_B, 1), jnp.float32),
        compiler_params=pltpu.CompilerParams(
            dimension_semantics=("arbitrary",)),
    )(u, i, c, *weights)


def kernel(user_id, item_id, item_catalog, user_table, item_table,
           catalog_table, lin_W, lin_b, cross_K, W1, b1, W2, b2, W3, b3,
           W4, b4, W5, b5):
    ids = [user_id.astype(jnp.int32) % _NB, item_id.astype(jnp.int32) % _NB,
           item_catalog.astype(jnp.int32) % _NB]
    srs, poss = [], []
    for r in ids:
        perm = jnp.argsort(r).astype(jnp.int32)
        srs.append(jnp.take(r, perm))
        poss.append(perm)
    sr_all = jnp.stack(srs).reshape(3 * _NW, _BPW)
    pos_all = jnp.stack(poss).reshape(3 * _NW, 4, 128)
    u_rows, i_rows, c_rows = _sc_gather(
        sr_all, pos_all, user_table.T, item_table.T, catalog_table.T)
    lwu = lin_W[0:64].reshape(1, _D)
    lwi = lin_W[64:128].reshape(1, _D)
    lwc = lin_W[128:192].reshape(1, _D)
    cku, cki, ckc = cross_K[0:64], cross_K[64:128], cross_K[128:192]
    w1u, w1i, w1c = W1[0:64], W1[64:128], W1[128:192]
    bias = (lin_b + b5).reshape(1, 1)
    return _tc_dense(u_rows, i_rows, c_rows, lwu, lwi, lwc, cku, cki, ckc,
                     w1u, w1i, w1c, b1.reshape(1, 256), W2, b2.reshape(1, 128),
                     W3, b3.reshape(1, 64), W4, b4.reshape(1, 32),
                     W5.reshape(1, 32), bias)


# speedup vs baseline: 1.1483x; 1.0774x over previous
import functools

import jax
import jax.numpy as jnp
from jax import lax
from jax.experimental import pallas as pl
from jax.experimental.pallas import tpu as pltpu
from jax.experimental.pallas import tpu_sc as plsc

_B = 16384
_D = 64
_NB = 1000000
_NC = 2
_NS = 16
_NW = _NC * _NS
_BPW = _B // _NW
_NG = _BPW // 16
_R = 8
_BM = 2048


def _sc_gather(sr_all, pos_all, ut_T, it_T, ct_T):
    mesh = plsc.VectorSubcoreMesh(core_axis_name="c", subcore_axis_name="s")

    @functools.partial(
        pl.kernel,
        out_type=(jax.ShapeDtypeStruct((_B, 128), jnp.float32),) * 3,
        mesh=mesh,
        scratch_types=[
            pltpu.VMEM((_BPW,), jnp.int32),
            pltpu.VMEM((4, 128), jnp.int32),
            pltpu.SMEM((_BPW,), jnp.int32),
            pltpu.VMEM((_R, 64, 128), jnp.float32),
            pltpu.VMEM((_BPW // 2, 128), jnp.float32),
            pltpu.SemaphoreType.DMA,
            pltpu.SemaphoreType.DMA((_R,)),
            pltpu.SemaphoreType.DMA,
        ],
        compiler_params=pltpu.CompilerParams(needs_layout_passes=False),
    )
    def gather(sr_hbm, pos_hbm, ut_hbm, it_hbm, ct_hbm, u_out, i_out, c_out,
               sr_v, pos_v, ucols_s, colbuf, rows, sem_i, sem_c, sem_o):
        wid = lax.axis_index("s") * _NC + lax.axis_index("c")
        for t, (tab, out) in enumerate(
            ((ut_hbm, u_out), (it_hbm, i_out), (ct_hbm, c_out))):
            pltpu.sync_copy(sr_hbm.at[t * _NW + wid], sr_v)
            pltpu.sync_copy(pos_hbm.at[t * _NW + wid], pos_v)

            def grp_a(g, carry):
                n_u, prev = carry
                cvec = lax.shift_right_logical(sr_v[pl.ds(g * 16, 16)], 7)
                for q in range(16):
                    c = cvec[q]
                    if q == 0:
                        chg = jnp.logical_or(g == 0, c != prev)
                    else:
                        chg = c != prev
                    @pl.when(chg)
                    def _():
                        ucols_s[n_u] = c
                    n_u = n_u + chg.astype(jnp.int32)
                    prev = c
                return n_u, prev

            n_u, _ = pl.loop(0, _NG, init_carry=(jnp.int32(0), jnp.int32(0)))(
                grp_a)

            for s in range(_R):
                @pl.when(s < n_u)
                def _():
                    c = ucols_s[s]
                    pltpu.async_copy(
                        tab.at[:, pl.ds(pl.multiple_of(c * 128, 128), 128)],
                        colbuf.at[s], sem_c.at[s])

            def mk_grp_b(mbase):
                def grp_b(g, carry):
                    u, prev = carry
                    vec = sr_v[pl.ds(g * 16, 16)]
                    cvec = lax.shift_right_logical(vec, 7)
                    lvec = lax.bitwise_and(vec, 127)
                    for q in range(16):
                        c = cvec[q]
                        l = lvec[q]
                        if q == 0:
                            chg = jnp.logical_or(
                                jnp.logical_and(g == 0, u < 0), c != prev)
                        else:
                            chg = c != prev
                        u = u + chg.astype(jnp.int32)

                        @pl.when(chg)
                        def _():
                            @pl.when(jnp.logical_and(u > 0, u - 1 + _R < n_u))
                            def _():
                                c2 = ucols_s[u - 1 + _R]
                                pltpu.async_copy(
                                    tab.at[:, pl.ds(
                                        pl.multiple_of(c2 * 128, 128), 128)],
                                    colbuf.at[lax.rem(u - 1, _R)],
                                    sem_c.at[lax.rem(u - 1, _R)])
                            slot_w = lax.rem(u, _R)
                            pltpu.make_async_copy(
                                tab.at[:, pl.ds(0, 128)], colbuf.at[slot_w],
                                sem_c.at[slot_w]).wait()

                        slot = lax.rem(u, _R)
                        m = g * 16 + q - mbase
                        lsplat = jnp.full((16,), l, jnp.int32)
                        for j in range(4):
                            d_idx = lax.iota(jnp.int32, 16) + (16 * j)
                            vals = plsc.load_gather(
                                colbuf.at[slot], [d_idx, lsplat])
                            rows[m, pl.ds(16 * j, 16)] = vals
                        prev = c
                    return u, prev
                return grp_b

            carry = (jnp.int32(-1), jnp.int32(0))
            for half in range(2):
                carry = pl.loop(half * (_NG // 2), (half + 1) * (_NG // 2),
                                init_carry=carry)(
                    mk_grp_b(half * (_BPW // 2)))
                cps = []
                for j in range(2):
                    cps.append(pltpu.async_copy(
                        rows.at[pl.ds(j * 128, 128)],
                        out.at[pos_v.at[half * 2 + j]], sem_o))
                for cp in cps:
                    cp.wait()

    return gather(sr_all, pos_all, ut_T, it_T, ct_T)


def _tc_dense(u, i, c, lwu, lwi, lwc, cku, cki, ckc,
              w1u, w1i, w1c, b1, w2, b2, w3, b3, w4, b4, w5, bias):
    f32 = jnp.float32

    def body(u_ref, i_ref, c_ref, lwu_ref, lwi_ref, lwc_ref,
             cku_ref, cki_ref, ckc_ref, w1u_ref, w1i_ref, w1c_ref, b1_ref,
             w2_ref, b2_ref, w3_ref, b3_ref, w4_ref, b4_ref, w5_ref,
             bias_ref, out_ref):
        uu = u_ref[:, :64]
        ii = i_ref[:, :64]
        cc = c_ref[:, :64]
        lin = jnp.sum(uu * lwu_ref[...] + ii * lwi_ref[...]
                      + cc * lwc_ref[...], axis=1, keepdims=True)
        xk = (jnp.dot(uu, cku_ref[...], preferred_element_type=f32)
              + jnp.dot(ii, cki_ref[...], preferred_element_type=f32)
              + jnp.dot(cc, ckc_ref[...], preferred_element_type=f32))
        a_sum = jnp.sum(xk * xk, axis=1, keepdims=True)
        bt = (jnp.dot(uu * uu, jnp.square(cku_ref[...]),
                      preferred_element_type=f32)
              + jnp.dot(ii * ii, jnp.square(cki_ref[...]),
                        preferred_element_type=f32)
              + jnp.dot(cc * cc, jnp.square(ckc_ref[...]),
                        preferred_element_type=f32))
        b_sum = jnp.sum(bt, axis=1, keepdims=True)
        fm = lin + (0.5 / 128.0) * (a_sum - b_sum)
        h = jax.nn.relu(jnp.dot(uu, w1u_ref[...], preferred_element_type=f32)
                        + jnp.dot(ii, w1i_ref[...], preferred_element_type=f32)
                        + jnp.dot(cc, w1c_ref[...], preferred_element_type=f32)
                        + b1_ref[...])
        h = jax.nn.relu(jnp.dot(h, w2_ref[...], preferred_element_type=f32)
                        + b2_ref[...])
        h = jax.nn.relu(jnp.dot(h, w3_ref[...], preferred_element_type=f32)
                        + b3_ref[...])
        h = jax.nn.relu(jnp.dot(h, w4_ref[...], preferred_element_type=f32)
                        + b4_ref[...])
        dnn = jnp.sum(h * w5_ref[...], axis=1, keepdims=True)
        out_ref[...] = jax.nn.sigmoid(fm + dnn + bias_ref[...])

    grid = (_B // _BM,)
    row_spec = pl.BlockSpec((_BM, 128), lambda m: (m, 0))

    def full(x):
        return pl.BlockSpec(x.shape, lambda m: (0,) * x.ndim)

    weights = (lwu, lwi, lwc, cku, cki, ckc, w1u, w1i, w1c, b1,
               w2, b2, w3, b3, w4, b4, w5, bias)
    return pl.pallas_call(
        body,
        grid=grid,
        in_specs=[row_spec, row_spec, row_spec] + [full(w) for w in weights],
        out_specs=pl.BlockSpec((_BM, 1), lambda m: (m, 0)),
        out_shape=jax.ShapeDtypeStruct((_B, 1), jnp.float32),
        compiler_params=pltpu.CompilerParams(
            dimension_semantics=("arbitrary",)),
    )(u, i, c, *weights)


def kernel(user_id, item_id, item_catalog, user_table, item_table,
           catalog_table, lin_W, lin_b, cross_K, W1, b1, W2, b2, W3, b3,
           W4, b4, W5, b5):
    ids = [user_id.astype(jnp.int32) % _NB, item_id.astype(jnp.int32) % _NB,
           item_catalog.astype(jnp.int32) % _NB]
    iota = lax.iota(jnp.int32, _B)
    srs, poss = [], []
    for r in ids:
        sr, perm = lax.sort_key_val(r, iota)
        srs.append(sr)
        poss.append(perm)
    sr_all = jnp.stack(srs).reshape(3 * _NW, _BPW)
    pos_all = jnp.stack(poss).reshape(3 * _NW, 4, 128)
    u_rows, i_rows, c_rows = _sc_gather(
        sr_all, pos_all, user_table.T, item_table.T, catalog_table.T)
    lwu = lin_W[0:64].reshape(1, _D)
    lwi = lin_W[64:128].reshape(1, _D)
    lwc = lin_W[128:192].reshape(1, _D)
    cku, cki, ckc = cross_K[0:64], cross_K[64:128], cross_K[128:192]
    w1u, w1i, w1c = W1[0:64], W1[64:128], W1[128:192]
    bias = (lin_b + b5).reshape(1, 1)
    return _tc_dense(u_rows, i_rows, c_rows, lwu, lwi, lwc, cku, cki, ckc,
                     w1u, w1i, w1c, b1.reshape(1, 256), W2, b2.reshape(1, 128),
                     W3, b3.reshape(1, 64), W4, b4.reshape(1, 32),
                     W5.reshape(1, 32), bias)


# speedup vs baseline: 1.1713x; 1.0201x over previous
import functools

import jax
import jax.numpy as jnp
from jax import lax
from jax.experimental import pallas as pl
from jax.experimental.pallas import tpu as pltpu
from jax.experimental.pallas import tpu_sc as plsc

_B = 16384
_D = 64
_NB = 1000000
_NC = 2
_NS = 16
_NW = _NC * _NS
_BPW = _B // _NW
_NG = _BPW // 16
_R = 10
_BM = 2048


def _sc_gather(sr_all, pos_all, ut_T, it_T, ct_T):
    mesh = plsc.VectorSubcoreMesh(core_axis_name="c", subcore_axis_name="s")

    @functools.partial(
        pl.kernel,
        out_type=(jax.ShapeDtypeStruct((_B, 128), jnp.float32),) * 3,
        mesh=mesh,
        scratch_types=[
            pltpu.VMEM((_BPW,), jnp.int32),
            pltpu.VMEM((_BPW,), jnp.int32),
            pltpu.VMEM((4, 128), jnp.int32),
            pltpu.VMEM((4, 128), jnp.int32),
            pltpu.SMEM((_BPW,), jnp.int32),
            pltpu.VMEM((_R, 64, 128), jnp.float32),
            pltpu.VMEM((_BPW // 2, 128), jnp.float32),
            pltpu.SemaphoreType.DMA,
            pltpu.SemaphoreType.DMA((_R,)),
            pltpu.SemaphoreType.DMA,
        ],
        compiler_params=pltpu.CompilerParams(needs_layout_passes=False),
    )
    def gather(sr_hbm, pos_hbm, ut_hbm, it_hbm, ct_hbm, u_out, i_out, c_out,
               sr_a, sr_b, pos_a, pos_b, ucols_s, colbuf, rows,
               sem_i, sem_c, sem_o):
        wid = lax.axis_index("s") * _NC + lax.axis_index("c")
        sr_bufs, pos_bufs = (sr_a, sr_b), (pos_a, pos_b)
        prev_cps = [
            pltpu.async_copy(sr_hbm.at[wid], sr_a, sem_i),
            pltpu.async_copy(pos_hbm.at[wid], pos_a, sem_i),
        ]
        for t, (tab, out) in enumerate(
            ((ut_hbm, u_out), (it_hbm, i_out), (ct_hbm, c_out))):
            sr_v = sr_bufs[t % 2]
            pos_v = pos_bufs[t % 2]
            for cp in prev_cps:
                cp.wait()
            if t < 2:
                prev_cps = [
                    pltpu.async_copy(sr_hbm.at[(t + 1) * _NW + wid],
                                     sr_bufs[(t + 1) % 2], sem_i),
                    pltpu.async_copy(pos_hbm.at[(t + 1) * _NW + wid],
                                     pos_bufs[(t + 1) % 2], sem_i),
                ]

            def grp_a(g, carry):
                n_u, prev = carry
                cvec = lax.shift_right_logical(sr_v[pl.ds(g * 16, 16)], 7)
                for q in range(16):
                    c = cvec[q]
                    if q == 0:
                        chg = jnp.logical_or(g == 0, c != prev)
                    else:
                        chg = c != prev
                    @pl.when(chg)
                    def _():
                        ucols_s[n_u] = c
                    n_u = n_u + chg.astype(jnp.int32)
                    prev = c
                return n_u, prev

            n_u, _ = pl.loop(0, _NG, init_carry=(jnp.int32(0), jnp.int32(0)))(
                grp_a)

            for s in range(_R):
                @pl.when(s < n_u)
                def _():
                    c = ucols_s[s]
                    pltpu.async_copy(
                        tab.at[:, pl.ds(pl.multiple_of(c * 128, 128), 128)],
                        colbuf.at[s], sem_c.at[s])

            def mk_grp_b(mbase):
                def grp_b(g, carry):
                    u, prev = carry
                    vec = sr_v[pl.ds(g * 16, 16)]
                    cvec = lax.shift_right_logical(vec, 7)
                    lvec = lax.bitwise_and(vec, 127)
                    for q in range(16):
                        c = cvec[q]
                        l = lvec[q]
                        if q == 0:
                            chg = jnp.logical_or(
                                jnp.logical_and(g == 0, u < 0), c != prev)
                        else:
                            chg = c != prev
                        u = u + chg.astype(jnp.int32)

                        @pl.when(chg)
                        def _():
                            @pl.when(jnp.logical_and(u > 0, u - 1 + _R < n_u))
                            def _():
                                c2 = ucols_s[u - 1 + _R]
                                pltpu.async_copy(
                                    tab.at[:, pl.ds(
                                        pl.multiple_of(c2 * 128, 128), 128)],
                                    colbuf.at[lax.rem(u - 1, _R)],
                                    sem_c.at[lax.rem(u - 1, _R)])
                            slot_w = lax.rem(u, _R)
                            pltpu.make_async_copy(
                                tab.at[:, pl.ds(0, 128)], colbuf.at[slot_w],
                                sem_c.at[slot_w]).wait()

                        slot = lax.rem(u, _R)
                        m = g * 16 + q - mbase
                        lsplat = jnp.full((16,), l, jnp.int32)
                        for j in range(4):
                            d_idx = lax.iota(jnp.int32, 16) + (16 * j)
                            vals = plsc.load_gather(
                                colbuf.at[slot], [d_idx, lsplat])
                            rows[m, pl.ds(16 * j, 16)] = vals
                        prev = c
                    return u, prev
                return grp_b

            carry = (jnp.int32(-1), jnp.int32(0))
            for half in range(2):
                carry = pl.loop(half * (_NG // 2), (half + 1) * (_NG // 2),
                                init_carry=carry)(
                    mk_grp_b(half * (_BPW // 2)))
                cps = []
                for j in range(2):
                    cps.append(pltpu.async_copy(
                        rows.at[pl.ds(j * 128, 128)],
                        out.at[pos_v.at[half * 2 + j]], sem_o))
                for cp in cps:
                    cp.wait()

    return gather(sr_all, pos_all, ut_T, it_T, ct_T)


def _tc_dense(u, i, c, lwu, lwi, lwc, cku, cki, ckc,
              w1u, w1i, w1c, b1, w2, b2, w3, b3, w4, b4, w5, bias):
    f32 = jnp.float32

    def body(u_ref, i_ref, c_ref, lwu_ref, lwi_ref, lwc_ref,
             cku_ref, cki_ref, ckc_ref, w1u_ref, w1i_ref, w1c_ref, b1_ref,
             w2_ref, b2_ref, w3_ref, b3_ref, w4_ref, b4_ref, w5_ref,
             bias_ref, out_ref):
        uu = u_ref[:, :64]
        ii = i_ref[:, :64]
        cc = c_ref[:, :64]
        lin = jnp.sum(uu * lwu_ref[...] + ii * lwi_ref[...]
                      + cc * lwc_ref[...], axis=1, keepdims=True)
        xk = (jnp.dot(uu, cku_ref[...], preferred_element_type=f32)
              + jnp.dot(ii, cki_ref[...], preferred_element_type=f32)
              + jnp.dot(cc, ckc_ref[...], preferred_element_type=f32))
        a_sum = jnp.sum(xk * xk, axis=1, keepdims=True)
        bt = (jnp.dot(uu * uu, jnp.square(cku_ref[...]),
                      preferred_element_type=f32)
              + jnp.dot(ii * ii, jnp.square(cki_ref[...]),
                        preferred_element_type=f32)
              + jnp.dot(cc * cc, jnp.square(ckc_ref[...]),
                        preferred_element_type=f32))
        b_sum = jnp.sum(bt, axis=1, keepdims=True)
        fm = lin + (0.5 / 128.0) * (a_sum - b_sum)
        h = jax.nn.relu(jnp.dot(uu, w1u_ref[...], preferred_element_type=f32)
                        + jnp.dot(ii, w1i_ref[...], preferred_element_type=f32)
                        + jnp.dot(cc, w1c_ref[...], preferred_element_type=f32)
                        + b1_ref[...])
        h = jax.nn.relu(jnp.dot(h, w2_ref[...], preferred_element_type=f32)
                        + b2_ref[...])
        h = jax.nn.relu(jnp.dot(h, w3_ref[...], preferred_element_type=f32)
                        + b3_ref[...])
        h = jax.nn.relu(jnp.dot(h, w4_ref[...], preferred_element_type=f32)
                        + b4_ref[...])
        dnn = jnp.sum(h * w5_ref[...], axis=1, keepdims=True)
        out_ref[...] = jax.nn.sigmoid(fm + dnn + bias_ref[...])

    grid = (_B // _BM,)
    row_spec = pl.BlockSpec((_BM, 128), lambda m: (m, 0))

    def full(x):
        return pl.BlockSpec(x.shape, lambda m: (0,) * x.ndim)

    weights = (lwu, lwi, lwc, cku, cki, ckc, w1u, w1i, w1c, b1,
               w2, b2, w3, b3, w4, b4, w5, bias)
    return pl.pallas_call(
        body,
        grid=grid,
        in_specs=[row_spec, row_spec, row_spec] + [full(w) for w in weights],
        out_specs=pl.BlockSpec((_BM, 1), lambda m: (m, 0)),
        out_shape=jax.ShapeDtypeStruct((_B, 1), jnp.float32),
        compiler_params=pltpu.CompilerParams(
            dimension_semantics=("arbitrary",)),
    )(u, i, c, *weights)


def kernel(user_id, item_id, item_catalog, user_table, item_table,
           catalog_table, lin_W, lin_b, cross_K, W1, b1, W2, b2, W3, b3,
           W4, b4, W5, b5):
    ids = [user_id.astype(jnp.int32) % _NB, item_id.astype(jnp.int32) % _NB,
           item_catalog.astype(jnp.int32) % _NB]
    iota = lax.iota(jnp.int32, _B)
    srs, poss = [], []
    for r in ids:
        sr, perm = lax.sort_key_val(r, iota)
        srs.append(sr)
        poss.append(perm)
    sr_all = jnp.stack(srs).reshape(3 * _NW, _BPW)
    pos_all = jnp.stack(poss).reshape(3 * _NW, 4, 128)
    u_rows, i_rows, c_rows = _sc_gather(
        sr_all, pos_all, user_table.T, item_table.T, catalog_table.T)
    lwu = lin_W[0:64].reshape(1, _D)
    lwi = lin_W[64:128].reshape(1, _D)
    lwc = lin_W[128:192].reshape(1, _D)
    cku, cki, ckc = cross_K[0:64], cross_K[64:128], cross_K[128:192]
    w1u, w1i, w1c = W1[0:64], W1[64:128], W1[128:192]
    bias = (lin_b + b5).reshape(1, 1)
    return _tc_dense(u_rows, i_rows, c_rows, lwu, lwi, lwc, cku, cki, ckc,
                     w1u, w1i, w1c, b1.reshape(1, 256), W2, b2.reshape(1, 128),
                     W3, b3.reshape(1, 64), W4, b4.reshape(1, 32),
                     W5.reshape(1, 32), bias)
